# baseline (speedup 1.0000x reference)
import functools

import jax
import jax.numpy as jnp
from jax import lax
from jax.experimental import pallas as pl
from jax.experimental.pallas import tpu as pltpu
from jax.experimental.pallas import tpu_sc as plsc

_N_USERS = 25000
_N_NODES = 50000
_D = 64
_E = 800000

_NC = 2
_NS = 16
_CHUNK = 96
_E_PAD = 804864
_NCH = _E_PAD // _CHUNK
_CH_PER_TILE = _NCH // _NS
_HALF = _N_NODES // _NC
_ACC_ROWS = _HALF + 88
_ZR = 32
_CP_ROWS = 1560


def _scale_chunk(ebuf, vbuf, gbuf, rloc, base_row):
    for g in range(_CHUNK // 16):
        sl16 = pl.ds(g * 16, 16)
        r = ebuf[1, sl16]
        loc = r - base_row
        oob = (loc < 0) | (loc >= _HALF)
        rloc[sl16] = jnp.where(oob, _HALF, loc)
        vv = vbuf[sl16]
        for k in range(16):
            i = g * 16 + k
            v = vv[k]
            for j in range(_D // 16):
                sl = pl.ds(j * 16, 16)
                gbuf[i, sl] = gbuf[i, sl] * v


def _prop_body(table, packed, valsh, out,
               ebuf0, ebuf1, ebuf2, ebuf3, vbuf0, vbuf1, vbuf2, vbuf3,
               gbuf0, gbuf1, gbuf2, gbuf3, rloc0, rloc1, rloc2, rloc3,
               zbuf, acc,
               se0, se1, se2, se3, sg0, sg1, sg2, sg3, ss0, ss1, ss2, ss3):
    core = lax.axis_index("c")
    sid = lax.axis_index("s")
    base_row = core * _HALF
    ebuf, vbuf = (ebuf0, ebuf1, ebuf2, ebuf3), (vbuf0, vbuf1, vbuf2, vbuf3)
    gbuf, rloc = (gbuf0, gbuf1, gbuf2, gbuf3), (rloc0, rloc1, rloc2, rloc3)
    se, sg, ss = (se0, se1, se2, se3), (sg0, sg1, sg2, sg3), (ss0, ss1, ss2, ss3)

    q0 = sid * _CH_PER_TILE

    def eload(c, b):
        qc = jnp.minimum(q0 + c, _NCH - 1)
        pltpu.async_copy(packed.at[qc], ebuf[b], se[b])
        pltpu.async_copy(valsh.at[pl.ds(qc * _CHUNK, _CHUNK)], vbuf[b], se[b])

    def ewait(b):
        pltpu.make_async_copy(packed.at[q0], ebuf[b], se[b]).wait()
        pltpu.make_async_copy(valsh.at[pl.ds(0, _CHUNK)], vbuf[b], se[b]).wait()

    def gather(b):
        pltpu.async_copy(table.at[ebuf[b].at[0]], gbuf[b], sg[b])

    def gwait(b):
        pltpu.make_async_copy(table.at[ebuf[b].at[0]], gbuf[b], sg[b]).wait()

    def swait(b):
        pltpu.make_async_copy(gbuf[b], acc.at[rloc[b]], ss[b]).wait()

    zero = jnp.zeros((16,), jnp.float32)
    for r in range(_ZR):
        for j in range(_D // 16):
            zbuf[r, pl.ds(j * 16, 16)] = zero

    def zloop(i, carry):
        pltpu.sync_copy(zbuf, acc.at[pl.ds(sid * 1568 + i * _ZR, _ZR)])
        return carry

    lax.fori_loop(0, 1568 // _ZR, zloop, 0)

    for b in range(4):
        eload(b, b)
    ewait(0)
    gather(0)
    ewait(1)
    gather(1)
    plsc.subcore_barrier()

    def chunk_quad(i, carry):
        for b in range(4):
            c = 4 * i + b
            gwait(b)
            _scale_chunk(ebuf[b], vbuf[b], gbuf[b], rloc[b], base_row)
            pltpu.async_copy(gbuf[b], acc.at[rloc[b]], ss[b], add=True)
            eload(c + 4, b)
            b2 = (b + 2) % 4
            ewait(b2)

            @pl.when(c >= 2)
            def _():
                swait(b2)

            gather(b2)
        return carry

    lax.fori_loop(0, _CH_PER_TILE // 4, chunk_quad, 0)

    swait(2)
    swait(3)
    gwait(0)
    gwait(1)
    ewait(2)
    ewait(3)
    plsc.subcore_barrier()

    pltpu.sync_copy(acc.at[pl.ds(sid * _CP_ROWS, _CP_ROWS)],
                    out.at[pl.ds(base_row + sid * _CP_ROWS, _CP_ROWS)])

    @pl.when(sid < 5)
    def _():
        tail = _NS * _CP_ROWS + sid * 8
        pltpu.sync_copy(acc.at[pl.ds(tail, 8)],
                        out.at[pl.ds(base_row + tail, 8)])


_prop = functools.partial(
    pl.kernel,
    mesh=plsc.VectorSubcoreMesh(core_axis_name="c", subcore_axis_name="s"),
    compiler_params=pltpu.CompilerParams(use_tc_tiling_on_sc=False),
    out_type=jax.ShapeDtypeStruct((_N_NODES, _D), jnp.float32),
    scratch_types=(
        [pltpu.VMEM((2, _CHUNK), jnp.int32) for _ in range(4)]
        + [pltpu.VMEM((_CHUNK,), jnp.float32) for _ in range(4)]
        + [pltpu.VMEM((_CHUNK, _D), jnp.float32) for _ in range(4)]
        + [pltpu.VMEM((_CHUNK,), jnp.int32) for _ in range(4)]
        + [pltpu.VMEM((_ZR, _D), jnp.float32),
           pltpu.VMEM_SHARED((_ACC_ROWS, _D), jnp.float32)]
        + [pltpu.SemaphoreType.DMA for _ in range(12)]
    ),
)(_prop_body)


def _mean_body(a, b, c, d, o):
    o[...] = (a[...] + b[...] + c[...] + d[...]) * 0.25


def _mean(x0, x1, x2, x3):
    blk = (1000, _D)
    spec = pl.BlockSpec(blk, lambda i: (i, 0))
    return pl.pallas_call(
        _mean_body,
        grid=(_N_NODES // blk[0],),
        in_specs=[spec] * 4,
        out_specs=spec,
        out_shape=jax.ShapeDtypeStruct((_N_NODES, _D), jnp.float32),
    )(x0, x1, x2, x3)


def kernel(user_emb, item_emb, edge_index, edge_values):
    rows = jnp.asarray(edge_index[0], jnp.int32)
    cols = jnp.asarray(edge_index[1], jnp.int32)
    vals = edge_values.astype(jnp.float32)
    pad = _E_PAD - _E
    rows = jnp.concatenate([rows, jnp.zeros((pad,), jnp.int32)])
    cols = jnp.concatenate([cols, jnp.zeros((pad,), jnp.int32)])
    vals = jnp.concatenate([vals, jnp.zeros((pad,), jnp.float32)])
    packed = jnp.stack([cols.reshape(_NCH, _CHUNK),
                        rows.reshape(_NCH, _CHUNK)], axis=1)

    x0 = jnp.concatenate([user_emb, item_emb], axis=0)
    x1 = _prop(x0, packed, vals)
    x2 = _prop(x1, packed, vals)
    x3 = _prop(x2, packed, vals)
    m = _mean(x0, x1, x2, x3)
    return m[:_N_USERS], m[_N_USERS:]
